# baseline (speedup 1.0000x reference)
import functools

import jax
import jax.numpy as jnp
from jax import lax
from jax.experimental import pallas as pl
from jax.experimental.pallas import tpu as pltpu
from jax.experimental.pallas import tpu_sc as plsc

NUM_EXPERTS = 64
TOP_K = 8
HIDDEN = 2048
TOKENS = 16384

TOKEN_BLOCK = 1024

NC = 2
NS = 16
LANES = 16
NW = NC * NS
TPW = TOKENS // NW


def _matmul_body(h_ref, w_ref, logits_ref, logits_t_ref):
    logits_ref[...] = lax.dot_general(
        h_ref[...], w_ref[...],
        dimension_numbers=(((1,), (1,)), ((), ())),
        preferred_element_type=jnp.float32,
    )
    logits_t_ref[...] = lax.dot_general(
        w_ref[...], h_ref[...],
        dimension_numbers=(((1,), (1,)), ((), ())),
        preferred_element_type=jnp.float32,
    )


def _tc_logits(hidden_states, gate_weight):
    n_blocks = TOKENS // TOKEN_BLOCK
    return pl.pallas_call(
        _matmul_body,
        grid=(n_blocks,),
        in_specs=[
            pl.BlockSpec((TOKEN_BLOCK, HIDDEN), lambda i: (i, 0)),
            pl.BlockSpec((NUM_EXPERTS, HIDDEN), lambda i: (0, 0)),
        ],
        out_specs=[
            pl.BlockSpec((TOKEN_BLOCK, NUM_EXPERTS), lambda i: (i, 0)),
            pl.BlockSpec((NUM_EXPERTS, TOKEN_BLOCK), lambda i: (0, i)),
        ],
        out_shape=[
            jax.ShapeDtypeStruct((TOKENS, NUM_EXPERTS), jnp.float32),
            jax.ShapeDtypeStruct((NUM_EXPERTS, TOKENS), jnp.float32),
        ],
        compiler_params=pltpu.CompilerParams(
            dimension_semantics=("arbitrary",),
        ),
    )(hidden_states, gate_weight)


def _sc_topk_body(lt_hbm, wt_hbm, et_hbm, lg_v, w_v, e_v):
    wid = lax.axis_index("s") * NC + lax.axis_index("c")
    base = wid * TPW
    pltpu.sync_copy(lt_hbm.at[:, pl.ds(base, TPW)], lg_v)

    neg_inf = jnp.full((LANES,), -jnp.inf, jnp.float32)
    zero_i = jnp.zeros((LANES,), jnp.int32)

    def group(g, _):
        t0 = g * LANES
        r = [neg_inf] * TOP_K
        ri = [zero_i] * TOP_K
        for e in range(NUM_EXPERTS):
            col = jnp.full((LANES,), e, jnp.int32)
            v = lg_v[e, pl.ds(t0, LANES)]
            c = [v > r[k] for k in range(TOP_K)]
            nr = [jnp.where(c[0], v, r[0])]
            nri = [jnp.where(c[0], col, ri[0])]
            for k in range(1, TOP_K):
                nr.append(jnp.where(c[k], jnp.where(c[k - 1], r[k - 1], v),
                                    r[k]))
                nri.append(jnp.where(c[k], jnp.where(c[k - 1], ri[k - 1], col),
                                     ri[k]))
            r, ri = nr, nri
        ex = [jnp.exp(r[k] - r[0]) for k in range(TOP_K)]
        s = ex[0]
        for k in range(1, TOP_K):
            s = s + ex[k]
        inv = 1.0 / s
        for k in range(TOP_K):
            w_v[k, pl.ds(t0, LANES)] = ex[k] * inv
            e_v[k, pl.ds(t0, LANES)] = ri[k]
        return ()

    lax.fori_loop(0, TPW // LANES, group, (), unroll=False)

    pltpu.sync_copy(w_v, wt_hbm.at[:, pl.ds(base, TPW)])
    pltpu.sync_copy(e_v, et_hbm.at[:, pl.ds(base, TPW)])


@functools.partial(
    pl.kernel,
    mesh=plsc.VectorSubcoreMesh(core_axis_name="c", subcore_axis_name="s"),
    out_type=[
        jax.ShapeDtypeStruct((TOP_K, TOKENS), jnp.float32),
        jax.ShapeDtypeStruct((TOP_K, TOKENS), jnp.int32),
    ],
    scratch_types=[
        pltpu.VMEM((NUM_EXPERTS, TPW), jnp.float32),
        pltpu.VMEM((TOP_K, TPW), jnp.float32),
        pltpu.VMEM((TOP_K, TPW), jnp.int32),
    ],
)
def _sc_topk(lt_hbm, wt_hbm, et_hbm, lg_v, w_v, e_v):
    _sc_topk_body(lt_hbm, wt_hbm, et_hbm, lg_v, w_v, e_v)


@jax.jit
def kernel(hidden_states, gate_weight):
    logits, logits_t = _tc_logits(hidden_states, gate_weight)
    weights_t, experts_t = _sc_topk(logits_t)
    return weights_t.T, experts_t.T, logits
